# baseline (speedup 1.0000x reference)
import jax
import jax.numpy as jnp
from jax import lax
from jax.experimental import pallas as pl
from jax.experimental.pallas import tpu as pltpu
from jax.experimental.pallas import tpu_sc as plsc

T = 1024
HD = 64
FD = 64
NF = 4
B = 8

NC = 2
NS = 16
L = 16
NW = NC * NS
NGRP = T // 8
GPW = NGRP // NW
TH = T // 2

RB = 64
CBJ = 256
CBQ = CBJ // 2

_NEG_BIG = -3.0e38


def _wh_body(h_ref, ww_ref, wb_ref, o_ref):
    o_ref[...] = lax.dot_general(
        h_ref[...], ww_ref[...],
        (((1,), (1,)), ((), ())),
        preferred_element_type=jnp.float32) + wb_ref[...]


def _s_body(a_ref, h_ref, o_ref):
    o_ref[...] = jnp.dot(a_ref[...], h_ref[...],
                         preferred_element_type=jnp.float32)


def _sigma_body(c0_ref, c1_ref, f_ref, wht_ref, oe_ref, oo_ref):
    rb = pl.program_id(0)
    cb = pl.program_id(1)
    needed = (cb >= c0_ref[rb]) & (cb < c1_ref[rb])

    @pl.when(needed)
    def _():
        ft = jnp.swapaxes(f_ref[...], 1, 2)
        prod = ft * wht_ref[...][None, :, :]
        oe_ref[...] = jnp.sum(prod[:, 0:FD, :], axis=1)
        oo_ref[...] = jnp.sum(prod[:, FD:2 * FD, :], axis=1)


def _sigma_call(c0s, c1s, fq, whqt):
    def fmap(rb, cb, c0, c1):
        cidx = jnp.clip(cb, c0[rb], jnp.maximum(c1[rb] - 1, c0[rb]))
        return rb, cidx, 0

    def whmap(rb, cb, c0, c1):
        cidx = jnp.clip(cb, c0[rb], jnp.maximum(c1[rb] - 1, c0[rb]))
        return 0, cidx

    omap = lambda rb, cb, c0, c1: (rb, cb)
    grid_spec = pltpu.PrefetchScalarGridSpec(
        num_scalar_prefetch=2,
        grid=(T // RB, T // CBJ),
        in_specs=[
            pl.BlockSpec((RB, CBQ, 2 * FD), fmap),
            pl.BlockSpec((2 * FD, CBQ), whmap),
        ],
        out_specs=[
            pl.BlockSpec((RB, CBQ), omap),
            pl.BlockSpec((RB, CBQ), omap),
        ],
    )
    return pl.pallas_call(
        _sigma_body,
        grid_spec=grid_spec,
        out_shape=[
            jax.ShapeDtypeStruct((T, TH), jnp.float32),
            jax.ShapeDtypeStruct((T, TH), jnp.float32),
        ],
    )(c0s, c1s, fq, whqt)


def _sc_body(sge_h, sgo_h, sb_h, ds_h, an_h, a_h,
             sb_v, sgeb, sgob, angb, dstb, attb,
             sem_e, sem_o, sem_a, sem_d):
    cid = lax.axis_index("c")
    sid = lax.axis_index("s")
    wid = sid * NC + cid

    pltpu.sync_copy(sb_h, sb_v)

    iota = lax.iota(jnp.int32, L)
    zeros16 = jnp.zeros((L,), jnp.float32)
    sbv = sb_v[0:2 * B]

    def grp_step(t, carry):
        grp = t * NW + wid
        i8 = pl.multiple_of(grp * 8, 8)

        cpe = pltpu.async_copy(sge_h.at[pl.ds(i8, 8)], sgeb, sem_e)
        cpo = pltpu.async_copy(sgo_h.at[pl.ds(i8, 8)], sgob, sem_o)
        cpa = pltpu.async_copy(an_h.at[pl.ds(i8, 8)], angb, sem_a)
        cpd = pltpu.async_copy(ds_h.at[pl.ds(i8, 8)], dstb, sem_d)
        cpe.wait()
        cpo.wait()
        cpa.wait()
        cpd.wait()

        for r8 in range(8):
            i = i8 + r8

            s = jnp.int32(0)
            e = jnp.int32(0)
            for b in range(B):
                sb = sbv[2 * b]
                eb = sbv[2 * b + 1]
                own = (sb <= i) & (i < eb) & ((eb - sb) > 1)
                s = jnp.where(own, sb, s)
                e = jnp.where(own, eb, e)

            g_lo = (s >> 1) >> 4
            g_hi = (((e + 1) >> 1) + L - 1) >> 4

            for gg in range(T // L):
                attb[r8, gg * L:(gg + 1) * L] = zeros16

            mx = jnp.float32(_NEG_BIG)
            kc = jnp.int32(0)
            for buf, base, off in ((sgeb, 0, 0), (sgob, TH, 1)):
                def mask_step(g2, mk, buf=buf, off=off):
                    mxx, kcc = mk
                    p0 = pl.multiple_of(g2 * L, 16)
                    jlan = 2 * (p0 + iota) + off
                    sig = buf[r8, pl.ds(p0, L)]
                    ang = angb[r8, pl.ds(base + p0, L)]
                    dst = dstb[r8, pl.ds(base + p0, L)]
                    msk = (ang < 0.0) | (dst > 10.0)
                    valid = (jlan >= s) & (jlan < e)
                    sigm = jnp.where((jlan == i) | msk, -1000.0, sig)
                    sigm = jnp.where(valid, sigm, _NEG_BIG)
                    buf[r8, pl.ds(p0, L)] = sigm
                    mxx = jnp.maximum(mxx, jnp.max(sigm))
                    kcc = kcc + jnp.sum(jnp.where(msk & valid, 1, 0))
                    return (mxx, kcc)

                mx, kc = lax.fori_loop(g_lo, g_hi, mask_step, (mx, kc))

            lsum = jnp.float32(0.0)
            for buf, base, off in ((sgeb, 0, 0), (sgob, TH, 1)):
                def exp_step(g2, ls, buf=buf, base=base):
                    p0 = pl.multiple_of(g2 * L, 16)
                    p = jnp.exp(buf[r8, pl.ds(p0, L)] - mx)
                    attb[r8, pl.ds(base + p0, L)] = p
                    return ls + jnp.sum(p)

                lsum = lax.fori_loop(g_lo, g_hi, exp_step, lsum)

            kzero = kc == (e - s - 1)
            lvec = jnp.full((L,), lsum, jnp.float32)
            scale = jnp.where(kzero, zeros16, 1.0 / lvec)

            for base in (0, TH):
                def scale_step(g2, _, base=base):
                    p0 = pl.multiple_of(g2 * L, 16)
                    attb[r8, pl.ds(base + p0, L)] = (
                        attb[r8, pl.ds(base + p0, L)] * scale)
                    return 0

                lax.fori_loop(g_lo, g_hi, scale_step, 0)

        pltpu.sync_copy(attb, a_h.at[pl.ds(i8, 8)])
        return carry

    lax.fori_loop(0, GPW, grp_step, 0)


def _owner_ranges(sub_batches):
    sb = sub_batches.astype(jnp.int32)
    idx = jnp.arange(T, dtype=jnp.int32)
    s_i = jnp.zeros((T,), jnp.int32)
    e_i = jnp.zeros((T,), jnp.int32)
    for b in range(B):
        own = (sb[b, 0] <= idx) & (idx < sb[b, 1]) & (sb[b, 1] - sb[b, 0] > 1)
        s_i = jnp.where(own, sb[b, 0], s_i)
        e_i = jnp.where(own, sb[b, 1], e_i)
    s_r = s_i.reshape(T // RB, RB)
    e_r = e_i.reshape(T // RB, RB)
    owned = e_r > 0
    smin = jnp.min(jnp.where(owned, s_r, T), axis=1)
    emax = jnp.max(e_r, axis=1)
    any_owned = jnp.any(owned, axis=1)
    c0s = jnp.where(any_owned, smin // CBJ, 0)
    c1s = jnp.where(any_owned, (emax + CBJ - 1) // CBJ, 0)
    return c0s, c1s


def kernel(f, h, sub_batches, features, hor_bearings_MTX, W_w, W_b):
    wh = pl.pallas_call(
        _wh_body,
        out_shape=jax.ShapeDtypeStruct((T, FD), jnp.float32),
    )(h, W_w, W_b.reshape(1, FD))

    c0s, c1s = _owner_ranges(sub_batches)
    zb = lax.optimization_barrier(jnp.zeros((1, 1, 1), jnp.float32))
    fq = f.reshape(T, TH, 2 * FD) + zb
    whqt = jnp.transpose(wh.reshape(TH, 2 * FD))
    sg_e, sg_o = _sigma_call(c0s, c1s, fq, whqt)

    perm = jnp.concatenate([jnp.arange(0, T, 2), jnp.arange(1, T, 2)])
    ang_p = hor_bearings_MTX[:, perm]
    dst_p = features[:, perm, 0]
    h_p = h[perm]

    mesh = plsc.VectorSubcoreMesh(core_axis_name="c", subcore_axis_name="s")
    a_p = pl.kernel(
        _sc_body,
        out_type=jax.ShapeDtypeStruct((T, T), jnp.float32),
        mesh=mesh,
        compiler_params=pltpu.CompilerParams(needs_layout_passes=False),
        scratch_types=[
            pltpu.VMEM((2 * B,), jnp.int32),
            pltpu.VMEM((8, TH), jnp.float32),
            pltpu.VMEM((8, TH), jnp.float32),
            pltpu.VMEM((8, T), jnp.float32),
            pltpu.VMEM((8, T), jnp.float32),
            pltpu.VMEM((8, T), jnp.float32),
            pltpu.SemaphoreType.DMA,
            pltpu.SemaphoreType.DMA,
            pltpu.SemaphoreType.DMA,
            pltpu.SemaphoreType.DMA,
        ],
    )(sg_e, sg_o, sub_batches.astype(jnp.int32).reshape(2 * B),
      dst_p, ang_p)

    return pl.pallas_call(
        _s_body,
        out_shape=jax.ShapeDtypeStruct((T, HD), jnp.float32),
    )(a_p, h_p)


# speedup vs baseline: 1.2713x; 1.2713x over previous
import jax
import jax.numpy as jnp
from jax import lax
from jax.experimental import pallas as pl
from jax.experimental.pallas import tpu as pltpu
from jax.experimental.pallas import tpu_sc as plsc

T = 1024
HD = 64
FD = 64
NF = 4
B = 8

NC = 2
NS = 16
L = 16
NW = NC * NS
NGRP = T // 8
GPW = NGRP // NW
TH = T // 2

RB = 64
CBJ = 256
CBQ = CBJ // 2

_NEG_BIG = -3.0e38


def _wh_body(h_ref, ww_ref, wb_ref, o_ref):
    o_ref[...] = lax.dot_general(
        h_ref[...], ww_ref[...],
        (((1,), (1,)), ((), ())),
        preferred_element_type=jnp.float32) + wb_ref[...]


def _s_body(a_ref, h_ref, o_ref):
    o_ref[...] = jnp.dot(a_ref[...], h_ref[...],
                         preferred_element_type=jnp.float32)


def _sigma_body(c0_ref, c1_ref, f_ref, wht_ref, oe_ref, oo_ref):
    rb = pl.program_id(0)
    cb = pl.program_id(1)
    needed = (cb >= c0_ref[rb]) & (cb < c1_ref[rb])

    @pl.when(needed)
    def _():
        ft = jnp.swapaxes(f_ref[...], 1, 2)
        prod = ft * wht_ref[...][None, :, :]
        oe_ref[...] = jnp.sum(prod[:, 0:FD, :], axis=1)
        oo_ref[...] = jnp.sum(prod[:, FD:2 * FD, :], axis=1)


def _sigma_call(c0s, c1s, fq, whqt):
    def fmap(rb, cb, c0, c1):
        cidx = jnp.clip(cb, c0[rb], jnp.maximum(c1[rb] - 1, c0[rb]))
        return rb, cidx, 0

    def whmap(rb, cb, c0, c1):
        cidx = jnp.clip(cb, c0[rb], jnp.maximum(c1[rb] - 1, c0[rb]))
        return 0, cidx

    omap = lambda rb, cb, c0, c1: (rb, cb)
    grid_spec = pltpu.PrefetchScalarGridSpec(
        num_scalar_prefetch=2,
        grid=(T // RB, T // CBJ),
        in_specs=[
            pl.BlockSpec((RB, CBQ, 2 * FD), fmap),
            pl.BlockSpec((2 * FD, CBQ), whmap),
        ],
        out_specs=[
            pl.BlockSpec((RB, CBQ), omap),
            pl.BlockSpec((RB, CBQ), omap),
        ],
    )
    return pl.pallas_call(
        _sigma_body,
        grid_spec=grid_spec,
        out_shape=[
            jax.ShapeDtypeStruct((T, TH), jnp.float32),
            jax.ShapeDtypeStruct((T, TH), jnp.float32),
        ],
    )(c0s, c1s, fq, whqt)


def _sc_body(sge_h, sgo_h, sb_h, ds_h, an_h, a_h,
             sb_v, sgeb, sgob, angb, dstb, attb,
             sem_e, sem_o, sem_a, sem_d):
    cid = lax.axis_index("c")
    sid = lax.axis_index("s")
    wid = sid * NC + cid

    pltpu.sync_copy(sb_h, sb_v)

    iota = lax.iota(jnp.int32, L)
    zeros16 = jnp.zeros((L,), jnp.float32)
    sbv = sb_v[0:2 * B]

    def grp_step(t, carry):
        grp = t * NW + wid
        i8 = pl.multiple_of(grp * 8, 8)

        cpe = pltpu.async_copy(sge_h.at[pl.ds(i8, 8)], sgeb, sem_e)
        cpo = pltpu.async_copy(sgo_h.at[pl.ds(i8, 8)], sgob, sem_o)
        cpa = pltpu.async_copy(an_h.at[pl.ds(i8, 8)], angb, sem_a)
        cpd = pltpu.async_copy(ds_h.at[pl.ds(i8, 8)], dstb, sem_d)
        cpe.wait()
        cpo.wait()
        cpa.wait()
        cpd.wait()

        for r8 in range(8):
            i = i8 + r8

            s = jnp.int32(0)
            e = jnp.int32(0)
            for b in range(B):
                sb = sbv[2 * b]
                eb = sbv[2 * b + 1]
                own = (sb <= i) & (i < eb) & ((eb - sb) > 1)
                s = jnp.where(own, sb, s)
                e = jnp.where(own, eb, e)

            g_lo = (s >> 1) >> 4
            g_hi = (((e + 1) >> 1) + L - 1) >> 4

            for gg in range(T // L):
                attb[r8, gg * L:(gg + 1) * L] = zeros16

            mx = jnp.float32(_NEG_BIG)
            kc = jnp.int32(0)
            for buf, base, off in ((sgeb, 0, 0), (sgob, TH, 1)):
                def mask_step(g2, mk, buf=buf, off=off):
                    mxx, kcc = mk
                    p0 = pl.multiple_of(g2 * L, 16)
                    jlan = 2 * (p0 + iota) + off
                    sig = buf[r8, pl.ds(p0, L)]
                    ang = angb[r8, pl.ds(base + p0, L)]
                    dst = dstb[r8, pl.ds(base + p0, L)]
                    msk = (ang < 0.0) | (dst > 10.0)
                    valid = (jlan >= s) & (jlan < e)
                    sigm = jnp.where((jlan == i) | msk, -1000.0, sig)
                    sigm = jnp.where(valid, sigm, _NEG_BIG)
                    buf[r8, pl.ds(p0, L)] = sigm
                    mxx = jnp.maximum(mxx, jnp.max(sigm))
                    kcc = kcc + jnp.sum(jnp.where(msk & valid, 1, 0))
                    return (mxx, kcc)

                mx, kc = lax.fori_loop(g_lo, g_hi, mask_step, (mx, kc))

            lsum = jnp.float32(0.0)
            for buf, base, off in ((sgeb, 0, 0), (sgob, TH, 1)):
                def exp_step(g2, ls, buf=buf, base=base):
                    p0 = pl.multiple_of(g2 * L, 16)
                    p = jnp.exp(buf[r8, pl.ds(p0, L)] - mx)
                    attb[r8, pl.ds(base + p0, L)] = p
                    return ls + jnp.sum(p)

                lsum = lax.fori_loop(g_lo, g_hi, exp_step, lsum)

            kzero = kc == (e - s - 1)
            lvec = jnp.full((L,), lsum, jnp.float32)
            scale = jnp.where(kzero, zeros16, 1.0 / lvec)

            for base in (0, TH):
                def scale_step(g2, _, base=base):
                    p0 = pl.multiple_of(g2 * L, 16)
                    attb[r8, pl.ds(base + p0, L)] = (
                        attb[r8, pl.ds(base + p0, L)] * scale)
                    return 0

                lax.fori_loop(g_lo, g_hi, scale_step, 0)

        pltpu.sync_copy(attb, a_h.at[pl.ds(i8, 8)])
        return carry

    lax.fori_loop(0, GPW, grp_step, 0)


def _owner_ranges(sub_batches):
    sb = sub_batches.astype(jnp.int32)
    idx = jnp.arange(T, dtype=jnp.int32)
    s_i = jnp.zeros((T,), jnp.int32)
    e_i = jnp.zeros((T,), jnp.int32)
    for b in range(B):
        own = (sb[b, 0] <= idx) & (idx < sb[b, 1]) & (sb[b, 1] - sb[b, 0] > 1)
        s_i = jnp.where(own, sb[b, 0], s_i)
        e_i = jnp.where(own, sb[b, 1], e_i)
    s_r = s_i.reshape(T // RB, RB)
    e_r = e_i.reshape(T // RB, RB)
    owned = e_r > 0
    smin = jnp.min(jnp.where(owned, s_r, T), axis=1)
    emax = jnp.max(e_r, axis=1)
    any_owned = jnp.any(owned, axis=1)
    c0s = jnp.where(any_owned, smin // CBJ, 0)
    c1s = jnp.where(any_owned, (emax + CBJ - 1) // CBJ, 0)
    return c0s, c1s


def kernel(f, h, sub_batches, features, hor_bearings_MTX, W_w, W_b):
    wh = pl.pallas_call(
        _wh_body,
        out_shape=jax.ShapeDtypeStruct((T, FD), jnp.float32),
    )(h, W_w, W_b.reshape(1, FD))

    c0s, c1s = _owner_ranges(sub_batches)
    fq = f.reshape(T, TH, 2 * FD)
    whqt = jnp.transpose(wh.reshape(TH, 2 * FD))
    sg_e, sg_o = _sigma_call(c0s, c1s, fq, whqt)

    perm = jnp.concatenate([jnp.arange(0, T, 2), jnp.arange(1, T, 2)])
    ang_p = hor_bearings_MTX[:, perm]
    dst_p = features[:, perm, 0]
    h_p = h[perm]

    mesh = plsc.VectorSubcoreMesh(core_axis_name="c", subcore_axis_name="s")
    a_p = pl.kernel(
        _sc_body,
        out_type=jax.ShapeDtypeStruct((T, T), jnp.float32),
        mesh=mesh,
        compiler_params=pltpu.CompilerParams(needs_layout_passes=False),
        scratch_types=[
            pltpu.VMEM((2 * B,), jnp.int32),
            pltpu.VMEM((8, TH), jnp.float32),
            pltpu.VMEM((8, TH), jnp.float32),
            pltpu.VMEM((8, T), jnp.float32),
            pltpu.VMEM((8, T), jnp.float32),
            pltpu.VMEM((8, T), jnp.float32),
            pltpu.SemaphoreType.DMA,
            pltpu.SemaphoreType.DMA,
            pltpu.SemaphoreType.DMA,
            pltpu.SemaphoreType.DMA,
        ],
    )(sg_e, sg_o, sub_batches.astype(jnp.int32).reshape(2 * B),
      dst_p, ang_p)

    return pl.pallas_call(
        _s_body,
        out_shape=jax.ShapeDtypeStruct((T, HD), jnp.float32),
    )(a_p, h_p)
